# Initial kernel scaffold; baseline (speedup 1.0000x reference)
import jax
import jax.numpy as jnp
from jax.experimental import pallas as pl


def kernel(x, W, L):
    raise NotImplementedError("write your pallas kernel here")


# speedup vs baseline: 3.2724x; 3.2724x over previous
import functools

import jax
import jax.numpy as jnp
from jax import lax
from jax.experimental import pallas as pl
from jax.experimental.pallas import tpu as pltpu

_GI, _GJ = 64, 128
_N = _GI * _GJ
_D = 32
_NL = 10
_BM = 256


def _som_block(x_ref, wt_ref, lp_ref, out_ref):
    xb = x_ref[...]
    wt = wt_ref[...]
    w2 = jnp.sum(wt * wt, axis=0, keepdims=True)
    xw = jnp.dot(xb, wt, preferred_element_type=jnp.float32)
    d2 = w2 - 2.0 * xw

    m = jnp.min(d2, axis=1, keepdims=True)
    idx = lax.broadcasted_iota(jnp.int32, d2.shape, 1)
    bmu = jnp.min(jnp.where(d2 <= m, idx, _N), axis=1, keepdims=True)

    pi = (bmu // _GJ).astype(jnp.float32)
    pj = (bmu % _GJ).astype(jnp.float32)

    gi = lax.broadcasted_iota(jnp.int32, (xb.shape[0], _GI), 1).astype(jnp.float32)
    gj = lax.broadcasted_iota(jnp.int32, (xb.shape[0], _GJ), 1).astype(jnp.float32)
    ai = jnp.exp(-0.5 * (pi - gi) ** 2)
    cj = jnp.exp(-0.5 * (pj - gj) ** 2)
    norm = jnp.sum(ai, axis=1, keepdims=True) * jnp.sum(cj, axis=1, keepdims=True)

    mm = jnp.dot(cj, lp_ref[...], preferred_element_type=jnp.float32)

    col = lax.broadcasted_iota(jnp.int32, (_GI, _GI * _NL), 1)
    row = lax.broadcasted_iota(jnp.int32, (_GI, _GI * _NL), 0)
    rmat = (col // _NL == row).astype(jnp.float32)
    scol = lax.broadcasted_iota(jnp.int32, (_GI * _NL, _NL), 0)
    srow = lax.broadcasted_iota(jnp.int32, (_GI * _NL, _NL), 1)
    smat = (scol % _NL == srow).astype(jnp.float32)

    ae = jnp.dot(ai, rmat, preferred_element_type=jnp.float32)
    out = jnp.dot(ae * mm, smat, preferred_element_type=jnp.float32)
    out_ref[...] = out / norm


@jax.jit
def kernel(x, W, L):
    B = x.shape[0]
    x = x.reshape(B, -1)
    wt = W.T
    lp = L.reshape(_GI, _GJ, _NL).transpose(1, 0, 2).reshape(_GJ, _GI * _NL)
    grid = (B // _BM,)
    return pl.pallas_call(
        _som_block,
        grid=grid,
        in_specs=[
            pl.BlockSpec((_BM, _D), lambda i: (i, 0)),
            pl.BlockSpec((_D, _N), lambda i: (0, 0)),
            pl.BlockSpec((_GJ, _GI * _NL), lambda i: (0, 0)),
        ],
        out_specs=pl.BlockSpec((_BM, _NL), lambda i: (i, 0)),
        out_shape=jax.ShapeDtypeStruct((B, _NL), jnp.float32),
        compiler_params=pltpu.CompilerParams(
            dimension_semantics=("parallel",),
        ),
    )(x, wt, lp)


# speedup vs baseline: 3.5418x; 1.0823x over previous
import functools

import jax
import jax.numpy as jnp
from jax import lax
from jax.experimental import pallas as pl
from jax.experimental.pallas import tpu as pltpu

_GI, _GJ = 64, 128
_N = _GI * _GJ
_D = 32
_NL = 10
_BM = 256


def _som_block(x_ref, wtn_ref, lp_ref, out_ref):
    xb = x_ref[...]
    wtn = wtn_ref[...]
    w2 = 0.25 * jnp.sum(wtn * wtn, axis=0, keepdims=True)
    kmat = jnp.concatenate([wtn, w2], axis=0)
    ones = jnp.ones((xb.shape[0], 1), dtype=jnp.float32)
    xb1 = jnp.concatenate([xb, ones], axis=1)
    k = jnp.dot(xb1, kmat, preferred_element_type=jnp.float32)

    m = jnp.min(k, axis=1, keepdims=True)
    idx = lax.broadcasted_iota(jnp.int32, k.shape, 1)
    bmu = jnp.min(jnp.where(k <= m, idx, _N), axis=1, keepdims=True)

    pi = (bmu // _GJ).astype(jnp.float32)
    pj = (bmu % _GJ).astype(jnp.float32)

    gi = lax.broadcasted_iota(jnp.int32, (xb.shape[0], _GI), 1).astype(jnp.float32)
    gj = lax.broadcasted_iota(jnp.int32, (xb.shape[0], _GJ), 1).astype(jnp.float32)
    ai = jnp.exp(-0.5 * (pi - gi) ** 2)
    cj = jnp.exp(-0.5 * (pj - gj) ** 2)
    norm = jnp.sum(ai, axis=1, keepdims=True) * jnp.sum(cj, axis=1, keepdims=True)

    mm = jnp.dot(cj, lp_ref[...], preferred_element_type=jnp.float32)

    col = lax.broadcasted_iota(jnp.int32, (_GI, _GI * _NL), 1)
    row = lax.broadcasted_iota(jnp.int32, (_GI, _GI * _NL), 0)
    rmat = (col // _NL == row).astype(jnp.float32)
    scol = lax.broadcasted_iota(jnp.int32, (_GI * _NL, _NL), 0)
    srow = lax.broadcasted_iota(jnp.int32, (_GI * _NL, _NL), 1)
    smat = (scol % _NL == srow).astype(jnp.float32)

    ae = jnp.dot(ai, rmat, preferred_element_type=jnp.float32)
    out = jnp.dot(ae * mm, smat, preferred_element_type=jnp.float32)
    out_ref[...] = out / norm


@jax.jit
def kernel(x, W, L):
    B = x.shape[0]
    x = x.reshape(B, -1)
    wt = (-2.0 * W).T
    lp = L.reshape(_GI, _GJ, _NL).transpose(1, 0, 2).reshape(_GJ, _GI * _NL)
    grid = (B // _BM,)
    return pl.pallas_call(
        _som_block,
        grid=grid,
        in_specs=[
            pl.BlockSpec((_BM, _D), lambda i: (i, 0)),
            pl.BlockSpec((_D, _N), lambda i: (0, 0)),
            pl.BlockSpec((_GJ, _GI * _NL), lambda i: (0, 0)),
        ],
        out_specs=pl.BlockSpec((_BM, _NL), lambda i: (i, 0)),
        out_shape=jax.ShapeDtypeStruct((B, _NL), jnp.float32),
        compiler_params=pltpu.CompilerParams(
            dimension_semantics=("parallel",),
        ),
    )(x, wt, lp)
